# Initial kernel scaffold; baseline (speedup 1.0000x reference)
import jax
import jax.numpy as jnp
from jax.experimental import pallas as pl


def kernel(variate_ids, value_num, cat_ids, variate_type, numeric_means, numeric_stds, w1, b1, w2, b2, cat_table, gamma, beta):
    raise NotImplementedError("write your pallas kernel here")


# speedup vs baseline: 2.1062x; 2.1062x over previous
import functools

import jax
import jax.numpy as jnp
from jax import lax
from jax.experimental import pallas as pl
from jax.experimental.pallas import tpu as pltpu
from jax.experimental.pallas import tpu_sc as plsc

D = 128
NV = 64
B = 1024
T = 200
P = B * T
NUM_CORES = 2
NUM_SUBCORES = 16
NW = NUM_CORES * NUM_SUBCORES
PW = P // NW
CHUNK = 128
NCHUNK = PW // CHUNK

TC_BLK = 1024
NBLK = P // TC_BLK


def _sc_gather(cid_hbm, table_hbm, ecat_hbm, idx_v, rows_v, gsem):
    wid = lax.axis_index("s") * NUM_CORES + lax.axis_index("c")
    base = wid * PW
    pltpu.sync_copy(cid_hbm.at[pl.ds(base, PW)], idx_v)

    def body(j, carry):
        off = j * CHUNK
        pltpu.async_copy(
            table_hbm.at[idx_v.at[pl.ds(off, CHUNK)]], rows_v, gsem
        ).wait()
        pltpu.sync_copy(rows_v, ecat_hbm.at[pl.ds(base + off, CHUNK)])
        return carry

    lax.fori_loop(0, NCHUNK, body, 0)


@functools.lru_cache(maxsize=None)
def _sc_gather_call():
    return pl.kernel(
        _sc_gather,
        out_type=jax.ShapeDtypeStruct((P, D), jnp.float32),
        mesh=plsc.VectorSubcoreMesh(
            core_axis_name="c", subcore_axis_name="s",
            num_cores=NUM_CORES, num_subcores=NUM_SUBCORES,
        ),
        scratch_types=[
            pltpu.VMEM((PW,), jnp.int32),
            pltpu.VMEM((CHUNK, D), jnp.float32),
            pltpu.SemaphoreType.DMA,
        ],
    )


def _tc_body(vid_ref, cid_ref, val_ref, ecat_ref, stats_ref, w1_ref, b1_ref,
             w2_ref, b2_ref, g_ref, be_ref, out_ref):
    vid = vid_ref[:, :]
    cid = cid_ref[:, :]
    val = val_ref[:, :]
    oh = (vid == lax.broadcasted_iota(jnp.int32, (TC_BLK, NV), 1))
    stats = jnp.dot(oh.astype(jnp.float32), stats_ref[:, :],
                    preferred_element_type=jnp.float32)
    mu = stats[:, 0:1]
    sg = stats[:, 1:2]
    ty = stats[:, 2:3]
    v = (val - mu) / (sg + 1e-6)
    h = jnp.maximum(v * w1_ref[0:1, :] + b1_ref[0:1, :], 0.0)
    e_num = jnp.dot(h, w2_ref[:, :], preferred_element_type=jnp.float32)
    e_num = e_num + b2_ref[0:1, :]
    mask_num = ty == 0.0
    mask_cat = jnp.logical_and(ty == 1.0, cid >= 0)
    e_val = jnp.where(mask_num, e_num, 0.0)
    e_val = jnp.where(mask_cat, ecat_ref[:, :], e_val)
    m = jnp.mean(e_val, axis=1, keepdims=True)
    d = e_val - m
    var = jnp.mean(d * d, axis=1, keepdims=True)
    out_ref[:, :] = d * lax.rsqrt(var + 1e-5) * g_ref[0:1, :] + be_ref[0:1, :]


def _small2d(shape):
    return pl.BlockSpec(shape, lambda i: (0,) * len(shape))


_tc_call = pl.pallas_call(
    _tc_body,
    grid=(NBLK,),
    in_specs=[
        pl.BlockSpec((TC_BLK, 1), lambda i: (i, 0)),
        pl.BlockSpec((TC_BLK, 1), lambda i: (i, 0)),
        pl.BlockSpec((TC_BLK, 1), lambda i: (i, 0)),
        pl.BlockSpec((TC_BLK, D), lambda i: (i, 0)),
        _small2d((NV, D)),
        _small2d((1, D)),
        _small2d((1, D)),
        _small2d((D, D)),
        _small2d((1, D)),
        _small2d((1, D)),
        _small2d((1, D)),
    ],
    out_specs=pl.BlockSpec((TC_BLK, D), lambda i: (i, 0)),
    out_shape=jax.ShapeDtypeStruct((P, D), jnp.float32),
)


def kernel(variate_ids, value_num, cat_ids, variate_type, numeric_means,
           numeric_stds, w1, b1, w2, b2, cat_table, gamma, beta):
    cid_flat = cat_ids.reshape(P)
    gather_ids = jnp.maximum(cid_flat, 0)

    e_cat = _sc_gather_call()(gather_ids, cat_table)

    stats_tab = (
        jnp.zeros((NV, D), jnp.float32)
        .at[:, 0].set(numeric_means)
        .at[:, 1].set(numeric_stds)
        .at[:, 2].set(variate_type.astype(jnp.float32))
    )
    w1p = jnp.zeros((1, D), jnp.float32).at[0, :16].set(w1)
    b1p = jnp.zeros((1, D), jnp.float32).at[0, :16].set(b1)
    w2p = jnp.zeros((D, D), jnp.float32).at[:16, :].set(w2)

    out = _tc_call(
        variate_ids.reshape(P, 1),
        cat_ids.reshape(P, 1),
        value_num.reshape(P, 1),
        e_cat,
        stats_tab,
        w1p,
        b1p,
        w2p,
        b2.reshape(1, D),
        gamma.reshape(1, D),
        beta.reshape(1, D),
    )
    return out.reshape(B, T, D)


# speedup vs baseline: 2.1076x; 1.0007x over previous
import functools

import jax
import jax.numpy as jnp
from jax import lax
from jax.experimental import pallas as pl
from jax.experimental.pallas import tpu as pltpu
from jax.experimental.pallas import tpu_sc as plsc

D = 128
NV = 64
B = 1024
T = 200
P = B * T
NUM_CORES = 2
NUM_SUBCORES = 16
NW = NUM_CORES * NUM_SUBCORES
PW = P // NW
CHUNK = 128
NCHUNK = PW // CHUNK

TC_BLK = 1024
NBLK = P // TC_BLK


def _sc_gather(cid_hbm, table_hbm, ecat_hbm, idx_v, rows_v, gsem, wsem):
    wid = lax.axis_index("s") * NUM_CORES + lax.axis_index("c")
    base = wid * PW
    pltpu.sync_copy(cid_hbm.at[pl.ds(base, PW)], idx_v)

    def g_start(j, slot):
        pltpu.async_copy(
            table_hbm.at[idx_v.at[pl.ds(j * CHUNK, CHUNK)]],
            rows_v.at[slot], gsem)

    def g_wait(slot):
        pltpu.make_async_copy(
            table_hbm.at[idx_v.at[pl.ds(0, CHUNK)]],
            rows_v.at[slot], gsem).wait()

    def w_start(j, slot):
        pltpu.async_copy(
            rows_v.at[slot], ecat_hbm.at[pl.ds(base + j * CHUNK, CHUNK)],
            wsem)

    def w_wait(slot):
        pltpu.make_async_copy(
            rows_v.at[slot], ecat_hbm.at[pl.ds(base, CHUNK)], wsem).wait()

    g_start(0, 0)
    g_start(1, 1)

    def body(j, carry):
        slot = lax.rem(j, 4)

        @pl.when(j >= 2)
        def _():
            w_wait(lax.rem(j - 2, 4))

        @pl.when(j + 2 < NCHUNK)
        def _():
            g_start(j + 2, lax.rem(j + 2, 4))

        g_wait(slot)
        w_start(j, slot)
        return carry

    lax.fori_loop(0, NCHUNK, body, 0)
    w_wait(lax.rem(NCHUNK - 2, 4))
    w_wait(lax.rem(NCHUNK - 1, 4))


@functools.lru_cache(maxsize=None)
def _sc_gather_call():
    return pl.kernel(
        _sc_gather,
        out_type=jax.ShapeDtypeStruct((P, D), jnp.float32),
        mesh=plsc.VectorSubcoreMesh(
            core_axis_name="c", subcore_axis_name="s",
            num_cores=NUM_CORES, num_subcores=NUM_SUBCORES,
        ),
        scratch_types=[
            pltpu.VMEM((PW,), jnp.int32),
            pltpu.VMEM((4, CHUNK, D), jnp.float32),
            pltpu.SemaphoreType.DMA,
            pltpu.SemaphoreType.DMA,
        ],
    )


def _tc_body(vid_ref, cid_ref, val_ref, ecat_ref, stats_ref, w1_ref, b1_ref,
             w2_ref, b2_ref, g_ref, be_ref, out_ref):
    vid = vid_ref[:, :]
    cid = cid_ref[:, :]
    val = val_ref[:, :]
    oh = (vid == lax.broadcasted_iota(jnp.int32, (TC_BLK, NV), 1))
    stats = jnp.dot(oh.astype(jnp.float32), stats_ref[:, :],
                    preferred_element_type=jnp.float32)
    mu = stats[:, 0:1]
    sg = stats[:, 1:2]
    ty = stats[:, 2:3]
    v = (val - mu) / (sg + 1e-6)
    h = jnp.maximum(v * w1_ref[0:1, :] + b1_ref[0:1, :], 0.0)
    e_num = jnp.dot(h, w2_ref[:, :], preferred_element_type=jnp.float32)
    e_num = e_num + b2_ref[0:1, :]
    mask_num = ty == 0.0
    mask_cat = jnp.logical_and(ty == 1.0, cid >= 0)
    e_val = jnp.where(mask_num, e_num, 0.0)
    e_val = jnp.where(mask_cat, ecat_ref[:, :], e_val)
    m = jnp.mean(e_val, axis=1, keepdims=True)
    d = e_val - m
    var = jnp.mean(d * d, axis=1, keepdims=True)
    out_ref[:, :] = d * lax.rsqrt(var + 1e-5) * g_ref[0:1, :] + be_ref[0:1, :]


def _small2d(shape):
    return pl.BlockSpec(shape, lambda i: (0,) * len(shape))


_tc_call = pl.pallas_call(
    _tc_body,
    grid=(NBLK,),
    in_specs=[
        pl.BlockSpec((TC_BLK, 1), lambda i: (i, 0)),
        pl.BlockSpec((TC_BLK, 1), lambda i: (i, 0)),
        pl.BlockSpec((TC_BLK, 1), lambda i: (i, 0)),
        pl.BlockSpec((TC_BLK, D), lambda i: (i, 0)),
        _small2d((NV, D)),
        _small2d((1, D)),
        _small2d((1, D)),
        _small2d((D, D)),
        _small2d((1, D)),
        _small2d((1, D)),
        _small2d((1, D)),
    ],
    out_specs=pl.BlockSpec((TC_BLK, D), lambda i: (i, 0)),
    out_shape=jax.ShapeDtypeStruct((P, D), jnp.float32),
)


def kernel(variate_ids, value_num, cat_ids, variate_type, numeric_means,
           numeric_stds, w1, b1, w2, b2, cat_table, gamma, beta):
    cid_flat = cat_ids.reshape(P)
    gather_ids = jnp.maximum(cid_flat, 0)

    e_cat = _sc_gather_call()(gather_ids, cat_table)

    stats_tab = (
        jnp.zeros((NV, D), jnp.float32)
        .at[:, 0].set(numeric_means)
        .at[:, 1].set(numeric_stds)
        .at[:, 2].set(variate_type.astype(jnp.float32))
    )
    w1p = jnp.zeros((1, D), jnp.float32).at[0, :16].set(w1)
    b1p = jnp.zeros((1, D), jnp.float32).at[0, :16].set(b1)
    w2p = jnp.zeros((D, D), jnp.float32).at[:16, :].set(w2)

    out = _tc_call(
        variate_ids.reshape(P, 1),
        cat_ids.reshape(P, 1),
        value_num.reshape(P, 1),
        e_cat,
        stats_tab,
        w1p,
        b1p,
        w2p,
        b2.reshape(1, D),
        gamma.reshape(1, D),
        beta.reshape(1, D),
    )
    return out.reshape(B, T, D)


# speedup vs baseline: 2.1912x; 1.0397x over previous
import functools

import jax
import jax.numpy as jnp
from jax import lax
from jax.experimental import pallas as pl
from jax.experimental.pallas import tpu as pltpu
from jax.experimental.pallas import tpu_sc as plsc

D = 128
NV = 64
B = 1024
T = 200
P = B * T
NUM_CORES = 2
NUM_SUBCORES = 16
NW = NUM_CORES * NUM_SUBCORES
PW = P // NW
CHUNK = 128
NCHUNK = PW // CHUNK

TC_BLK = 1024
NBLK = P // TC_BLK


def _sc_gather(cid_hbm, table_hbm, ecat_hbm, idx_v, rows_v, gsem, wsem):
    wid = lax.axis_index("s") * NUM_CORES + lax.axis_index("c")
    base = wid * PW
    pltpu.sync_copy(cid_hbm.at[pl.ds(base, PW)], idx_v)

    def g_start(j, slot):
        pltpu.async_copy(
            table_hbm.at[idx_v.at[pl.ds(j * CHUNK, CHUNK)]],
            rows_v.at[slot], gsem)

    def g_wait(slot):
        pltpu.make_async_copy(
            table_hbm.at[idx_v.at[pl.ds(0, CHUNK)]],
            rows_v.at[slot], gsem).wait()

    def w_start(j, slot):
        pltpu.async_copy(
            rows_v.at[slot], ecat_hbm.at[pl.ds(base + j * CHUNK, CHUNK)],
            wsem)

    def w_wait(slot):
        pltpu.make_async_copy(
            rows_v.at[slot], ecat_hbm.at[pl.ds(base, CHUNK)], wsem).wait()

    g_start(0, 0)
    g_start(1, 1)

    def body(j, carry):
        slot = lax.rem(j, 4)

        @pl.when(j >= 2)
        def _():
            w_wait(lax.rem(j - 2, 4))

        @pl.when(j + 2 < NCHUNK)
        def _():
            g_start(j + 2, lax.rem(j + 2, 4))

        g_wait(slot)
        w_start(j, slot)
        return carry

    lax.fori_loop(0, NCHUNK, body, 0)
    w_wait(lax.rem(NCHUNK - 2, 4))
    w_wait(lax.rem(NCHUNK - 1, 4))


@functools.lru_cache(maxsize=None)
def _sc_gather_call():
    return pl.kernel(
        _sc_gather,
        out_type=jax.ShapeDtypeStruct((P, D), jnp.float32),
        mesh=plsc.VectorSubcoreMesh(
            core_axis_name="c", subcore_axis_name="s",
            num_cores=NUM_CORES, num_subcores=NUM_SUBCORES,
        ),
        scratch_types=[
            pltpu.VMEM((PW,), jnp.int32),
            pltpu.VMEM((4, CHUNK, D), jnp.float32),
            pltpu.SemaphoreType.DMA,
            pltpu.SemaphoreType.DMA,
        ],
    )


def _tc_body(vidf_ref, cidf_ref, val_ref, ecat_ref, ones64_ref, m1_ref,
             m2_ref, ab2_ref, bc_ref, w2_ref, invd_ref, g_ref, be_ref,
             out_ref):
    vid = vidf_ref[:, :]
    cid = cidf_ref[:, :]
    val = val_ref[:, :]
    ones64 = ones64_ref[:, :]
    f32 = jnp.float32
    dot = lambda a, b: jnp.dot(a, b, preferred_element_type=f32)
    vidb = dot(vid, ones64)
    valb = dot(val, ones64)
    cidb = dot(cid, ones64)
    eq = vidb == lax.broadcasted_iota(jnp.int32, (TC_BLK, NV), 1).astype(f32)
    oh = jnp.where(eq, 1.0, 0.0)
    ohv = jnp.where(eq, valb, 0.0)
    ohc = jnp.where(jnp.logical_and(eq, cidb >= 0.0), 1.0, 0.0)
    h = jnp.maximum(dot(ohv, m1_ref[:, :]) + dot(oh, m2_ref[:, :]), 0.0)
    e_val = (dot(h, w2_ref[:, :]) + dot(oh, ab2_ref[:, :])
             + ecat_ref[:, :] * dot(ohc, bc_ref[:, :]))
    m = dot(e_val, invd_ref[:, 0:1])
    s2 = dot(e_val * e_val, invd_ref[:, 1:2])
    r = lax.rsqrt(s2 - m * m + 1e-5)
    rb = dot(r, g_ref[:, :])
    cb = dot(-m * r, g_ref[:, :])
    out_ref[:, :] = e_val * rb + (cb + be_ref[0:1, :])


def _small2d(shape):
    return pl.BlockSpec(shape, lambda i: (0,) * len(shape))


def _tc_specs():
    return [
        pl.BlockSpec((TC_BLK, 1), lambda i: (i, 0)),
        pl.BlockSpec((TC_BLK, 1), lambda i: (i, 0)),
        pl.BlockSpec((TC_BLK, 1), lambda i: (i, 0)),
        pl.BlockSpec((TC_BLK, D), lambda i: (i, 0)),
        _small2d((1, NV)),
        _small2d((NV, D)),
        _small2d((NV, D)),
        _small2d((NV, D)),
        _small2d((NV, D)),
        _small2d((D, D)),
        _small2d((D, 2)),
        _small2d((1, D)),
        _small2d((1, D)),
    ]


def _tc_forward(e_cat, variate_ids, value_num, cat_ids, variate_type,
                numeric_means, numeric_stds, w1, b1, w2, b2, gamma, beta,
                interpret=False):
    f32 = jnp.float32
    isg = 1.0 / (numeric_stds + 1e-6)
    misg = numeric_means * isg
    an = (variate_type == 0).astype(f32)[:, None]
    ac = (variate_type == 1).astype(f32)[:, None]
    w1row = jnp.zeros((1, D), f32).at[0, :16].set(w1)
    b1row = jnp.zeros((1, D), f32).at[0, :16].set(b1)
    m1 = an * isg[:, None] * w1row
    m2 = an * (-misg[:, None] * w1row + b1row)
    ab2 = an * b2[None, :]
    bc = ac * jnp.ones((1, D), f32)
    w2p = jnp.zeros((D, D), f32).at[:16, :].set(w2)
    call = pl.pallas_call(
        _tc_body,
        grid=(NBLK,),
        in_specs=_tc_specs(),
        out_specs=pl.BlockSpec((TC_BLK, D), lambda i: (i, 0)),
        out_shape=jax.ShapeDtypeStruct((P, D), f32),
        interpret=interpret,
    )
    return call(
        variate_ids.astype(f32).reshape(P, 1),
        cat_ids.astype(f32).reshape(P, 1),
        value_num.reshape(P, 1),
        e_cat,
        jnp.ones((1, NV), f32),
        m1, m2, ab2, bc, w2p,
        jnp.full((D, 2), 1.0 / D, f32),
        gamma.reshape(1, D),
        beta.reshape(1, D),
    )


def kernel(variate_ids, value_num, cat_ids, variate_type, numeric_means,
           numeric_stds, w1, b1, w2, b2, cat_table, gamma, beta):
    cid_flat = cat_ids.reshape(P)
    gather_ids = jnp.maximum(cid_flat, 0)

    e_cat = _sc_gather_call()(gather_ids, cat_table)

    out = _tc_forward(e_cat, variate_ids, value_num, cat_ids, variate_type,
                      numeric_means, numeric_stds, w1, b1, w2, b2, gamma,
                      beta)
    return out.reshape(B, T, D)
